# baseline (speedup 1.0000x reference)
import functools

import jax
import jax.numpy as jnp
from jax import lax
from jax.experimental import pallas as pl
from jax.experimental.pallas import tpu as pltpu
from jax.experimental.pallas import tpu_sc as plsc


def _tv_body(nv_ref, o_ref):
    x = nv_ref[...]
    o_ref[...] = jnp.sign(x) * jnp.log1p(jnp.abs(x))


_G = 128
_NBUF = 5
_DEPTH = 3


def _make_sc_gather(N, H, n_per_tile):
    n_chunks = n_per_tile // _G
    assert n_chunks % _NBUF == 0
    n_super = n_chunks // _NBUF
    mesh = plsc.VectorSubcoreMesh(core_axis_name="c", subcore_axis_name="s")

    @functools.partial(
        pl.kernel,
        out_type=jax.ShapeDtypeStruct((N, H), jnp.float32),
        mesh=mesh,
        scratch_types=[
            pltpu.VMEM((n_per_tile,), jnp.int32),
            pltpu.VMEM((n_per_tile,), jnp.float32),
            pltpu.VMEM((H,), jnp.float32),
            pltpu.VMEM((_NBUF, _G, H), jnp.float32),
            [pltpu.SemaphoreType.DMA] * _NBUF,
            [pltpu.SemaphoreType.DMA] * _NBUF,
        ],
        compiler_params=pltpu.CompilerParams(use_tc_tiling_on_sc=False),
    )
    def sc_gather(emb_hbm, idx_hbm, tv_hbm, dir_hbm, out_hbm,
                  idx_v, tv_v, dir_v, rows_v, gsems, osems):
        wid = lax.axis_index("s") * mesh.num_cores + lax.axis_index("c")
        tile_base = wid * n_per_tile

        pltpu.sync_copy(idx_hbm.at[pl.ds(tile_base, n_per_tile)], idx_v)
        pltpu.sync_copy(tv_hbm.at[pl.ds(tile_base, n_per_tile)], tv_v)
        pltpu.sync_copy(dir_hbm, dir_v)
        dvecs = [dir_v[pl.ds(16 * k, 16)] for k in range(H // 16)]

        def fire_gather(g, b):
            pltpu.async_copy(
                emb_hbm.at[idx_v.at[pl.ds(g * _G, _G)]], rows_v.at[b],
                gsems[b])

        def wait_gather(b):
            pltpu.make_async_copy(
                emb_hbm.at[pl.ds(0, _G)], rows_v.at[b], gsems[b]).wait()

        def fire_write(g, b):
            pltpu.async_copy(
                rows_v.at[b], out_hbm.at[pl.ds(tile_base + g * _G, _G)],
                osems[b])

        def wait_write(b):
            pltpu.make_async_copy(
                rows_v.at[b], out_hbm.at[pl.ds(0, _G)], osems[b]).wait()

        for b in range(_DEPTH):
            fire_gather(b, b)

        def super_step(s, _):
            for b in range(_NBUF):
                g = s * _NBUF + b
                wait_gather(b)
                goff = g * _G

                def grp_body(j, _):
                    tvec = tv_v[pl.ds(goff + 16 * j, 16)]
                    base = 16 * j
                    for jj in range(16):
                        t = tvec[jj]
                        i = base + jj
                        for k in range(H // 16):
                            sl = pl.ds(16 * k, 16)
                            rows_v[b, i, sl] = rows_v[b, i, sl] + t * dvecs[k]
                    return 0

                lax.fori_loop(0, _G // 16, grp_body, 0)
                fire_write(g, b)

                pb = (b + _DEPTH) % _NBUF

                @pl.when(g + _DEPTH < n_chunks)
                def _():
                    @pl.when(g >= _NBUF - _DEPTH)
                    def _():
                        wait_write(pb)
                    fire_gather(g + _DEPTH, pb)
            return 0

        lax.fori_loop(0, n_super, super_step, 0)

        for b in range(_NBUF):
            wait_write(b)

    return sc_gather


def kernel(input_ids, numerical_values, embedding, numerical_direction):
    B, S = input_ids.shape
    V, H = embedding.shape
    N = B * S
    ids = input_ids.reshape(N).astype(jnp.int32)
    nv = numerical_values.reshape(N // 128, 128)

    tv = pl.pallas_call(
        _tv_body,
        out_shape=jax.ShapeDtypeStruct((N // 128, 128), jnp.float32),
    )(nv).reshape(N)

    nw = 32
    n_per_tile = N // nw
    out = _make_sc_gather(N, H, n_per_tile)(
        embedding, ids, tv, numerical_direction)
    return out.reshape(B, S, H)
